# baseline (speedup 1.0000x reference)
import functools

import jax
import jax.numpy as jnp
from jax import lax
from jax.experimental import pallas as pl
from jax.experimental.pallas import tpu as pltpu
from jax.experimental.pallas import tpu_sc as plsc

N = 10000
E = 320000
D = 128
G = 16
NP = 10240

NC = 2
NS = 16
NW = NC * NS
L = 16

EPW = E // NW
DCH = 80
DNCH = EPW // DCH
RPT = NP // NS

CH = 64
NCH = 320
NCHH = NCH // 2
EPAD = NS * NCH * CH
HALF = NP // NC
TRASH = HALF
HALFP = HALF + CH
RPC = HALF // NS


def _deg_body(dst_hbm, out_hbm, didx, hist, outst, red, shared):
    c = lax.axis_index("c")
    s = lax.axis_index("s")
    w = s * NC + c
    pltpu.sync_copy(dst_hbm.at[w], didx)

    zero16 = jnp.zeros((L,), jnp.float32)
    ones16 = jnp.full((L,), 1.0, jnp.float32)

    def _zero(i, carry):
        hist[pl.ds(i * L, L)] = zero16
        return carry
    lax.fori_loop(0, NP // L, _zero, 0)

    def _count(r, carry):
        def _inner(cc, carry2):
            idx = didx[r, pl.ds(cc * L, L)]
            plsc.addupdate_scatter(hist, [idx], ones16)
            return carry2
        return lax.fori_loop(0, DCH // L, _inner, carry)
    lax.fori_loop(0, DNCH, _count, 0)

    pltpu.sync_copy(hist, shared.at[s])
    plsc.subcore_barrier()
    base = s * RPT
    pltpu.sync_copy(shared.at[:, pl.ds(base, RPT)], red)

    def _reduce(j, carry):
        acc = jnp.zeros((L,), jnp.float32)
        for r in range(NS):
            acc = acc + red[r, pl.ds(j * L, L)]
        outst[pl.ds(j * L, L)] = acc
        return carry
    lax.fori_loop(0, RPT // L, _reduce, 0)
    pltpu.sync_copy(outst, out_hbm.at[c, pl.ds(base, RPT)])


def _agg_body(src_hbm, dst_hbm, tab_hbm, out_hbm,
              sidx, didx, rows_a, rows_b, rows_c, rows_d, acc,
              sem_a, sem_b, sem_c, sem_d):
    c = lax.axis_index("c")
    s = lax.axis_index("s")
    base = s * RPC

    zero16 = jnp.zeros((L,), jnp.float32)

    def _zfill(i, carry):
        r = i // (D // L)
        col = (i % (D // L)) * L
        rows_a[r, pl.ds(col, L)] = zero16
        return carry
    lax.fori_loop(0, CH * (D // L), _zfill, 0)

    for i in range(RPC // CH):
        pltpu.sync_copy(rows_a, acc.at[pl.ds(base + i * CH, CH)])

    @pl.when(s == 0)
    def _():
        pltpu.sync_copy(rows_a, acc.at[pl.ds(HALF, CH)])

    plsc.subcore_barrier()

    lo = c * HALF

    for p in range(2):
        pltpu.sync_copy(src_hbm.at[s, pl.ds(p * NCHH, NCHH)], sidx)
        pltpu.sync_copy(dst_hbm.at[s, pl.ds(p * NCHH, NCHH)], didx)

        def _remap(r, carry):
            for cc in range(CH // L):
                g = didx[r, pl.ds(cc * L, L)] - lo
                ok = (g >= 0) & (g < HALF)
                didx[r, pl.ds(cc * L, L)] = jnp.where(ok, g, TRASH)
            return carry
        lax.fori_loop(0, NCHH, _remap, 0)

        pltpu.async_copy(tab_hbm.at[sidx.at[0]], rows_a, sem_a)
        pltpu.async_copy(tab_hbm.at[sidx.at[1]], rows_b, sem_b)
        pltpu.async_copy(tab_hbm.at[sidx.at[2]], rows_c, sem_c)

        def _quad(t, carry):
            j = 4 * t
            pltpu.make_async_copy(tab_hbm.at[sidx.at[j]], rows_a, sem_a).wait()
            pltpu.async_copy(tab_hbm.at[sidx.at[j + 3]], rows_d, sem_d)
            pltpu.sync_copy(rows_a, acc.at[didx.at[j]], add=True)

            pltpu.make_async_copy(tab_hbm.at[sidx.at[j + 1]], rows_b,
                                  sem_b).wait()
            @pl.when(j + 4 < NCHH)
            def _():
                pltpu.async_copy(tab_hbm.at[sidx.at[j + 4]], rows_a, sem_a)
            pltpu.sync_copy(rows_b, acc.at[didx.at[j + 1]], add=True)

            pltpu.make_async_copy(tab_hbm.at[sidx.at[j + 2]], rows_c,
                                  sem_c).wait()
            @pl.when(j + 5 < NCHH)
            def _():
                pltpu.async_copy(tab_hbm.at[sidx.at[j + 5]], rows_b, sem_b)
            pltpu.sync_copy(rows_c, acc.at[didx.at[j + 2]], add=True)

            pltpu.make_async_copy(tab_hbm.at[sidx.at[j + 3]], rows_d,
                                  sem_d).wait()
            @pl.when(j + 6 < NCHH)
            def _():
                pltpu.async_copy(tab_hbm.at[sidx.at[j + 6]], rows_c, sem_c)
            pltpu.sync_copy(rows_d, acc.at[didx.at[j + 3]], add=True)
            return carry
        lax.fori_loop(0, NCHH // 4, _quad, 0)

    plsc.subcore_barrier()
    pltpu.sync_copy(acc.at[pl.ds(base, RPC)],
                    out_hbm.at[pl.ds(c * HALF + base, RPC)])


BLK = 2048
NBLK = NP // BLK


def _prep_body(x_ref, w1_ref, p0_ref, p1_ref, o_ref):
    dinv = lax.rsqrt(1.0 + p0_ref[...] + p1_ref[...])
    h = jnp.dot(x_ref[...], w1_ref[...], preferred_element_type=jnp.float32)
    o_ref[...] = h * dinv


def _mid_body(a_ref, t1_ref, p0_ref, p1_ref, b1_ref, w2_ref, o_ref):
    dinv = lax.rsqrt(1.0 + p0_ref[...] + p1_ref[...])
    z = dinv * (a_ref[...] + t1_ref[...]) + b1_ref[...]
    z = jnp.maximum(z, 0.0)
    o_ref[...] = jnp.dot(z, w2_ref[...], preferred_element_type=jnp.float32) * dinv


def _fin_body(a_ref, t2_ref, p0_ref, p1_ref, b2_ref, br_ref,
              wc_ref, bc_ref, o_ref, sums, cnt):
    i = pl.program_id(0)

    @pl.when(i == 0)
    def _():
        sums[...] = jnp.zeros_like(sums)
        cnt[...] = jnp.zeros_like(cnt)

    dinv = lax.rsqrt(1.0 + p0_ref[...] + p1_ref[...])
    z = dinv * (a_ref[...] + t2_ref[...]) + b2_ref[...]
    ids = lax.broadcasted_iota(jnp.int32, (G, BLK), 0)
    oh = (ids == br_ref[...]).astype(jnp.float32)
    sums[...] += jnp.dot(oh, z, preferred_element_type=jnp.float32)
    cnt[...] += jnp.sum(oh, axis=1, keepdims=True)

    @pl.when(i == pl.num_programs(0) - 1)
    def _():
        pooled = sums[...] / jnp.maximum(cnt[...], 1.0)
        o_ref[...] = (jnp.dot(pooled, wc_ref[...],
                              preferred_element_type=jnp.float32) + bc_ref[...])


def _row_spec(width):
    return pl.BlockSpec((BLK, width), lambda i: (i, 0))


def _const_spec(shape):
    return pl.BlockSpec(shape, lambda i: tuple(0 for _ in shape))


@functools.lru_cache(maxsize=2)
def _tc_kernels(interpret: bool = False):
    prep = pl.pallas_call(
        _prep_body,
        grid=(NBLK,),
        in_specs=[_row_spec(D), _const_spec((D, D)), _row_spec(1),
                  _row_spec(1)],
        out_specs=_row_spec(D),
        out_shape=jax.ShapeDtypeStruct((NP, D), jnp.float32),
        interpret=interpret,
    )
    mid = pl.pallas_call(
        _mid_body,
        grid=(NBLK,),
        in_specs=[_row_spec(D), _row_spec(D), _row_spec(1), _row_spec(1),
                  _const_spec((1, D)), _const_spec((D, D))],
        out_specs=_row_spec(D),
        out_shape=jax.ShapeDtypeStruct((NP, D), jnp.float32),
        interpret=interpret,
    )
    fin = pl.pallas_call(
        _fin_body,
        grid=(NBLK,),
        in_specs=[_row_spec(D), _row_spec(D), _row_spec(1), _row_spec(1),
                  _const_spec((1, D)),
                  pl.BlockSpec((1, BLK), lambda i: (0, i)),
                  _const_spec((D, 2)), _const_spec((1, 2))],
        out_specs=_const_spec((G, 2)),
        out_shape=jax.ShapeDtypeStruct((G, 2), jnp.float32),
        scratch_shapes=[pltpu.VMEM((G, D), jnp.float32),
                        pltpu.VMEM((G, 1), jnp.float32)],
        interpret=interpret,
    )
    return prep, mid, fin


@functools.lru_cache(maxsize=1)
def _sc_kernels():
    mesh = plsc.VectorSubcoreMesh(core_axis_name="c", subcore_axis_name="s",
                                  num_cores=NC, num_subcores=NS)
    deg = pl.kernel(
        _deg_body,
        out_type=jax.ShapeDtypeStruct((NC, NP), jnp.float32),
        mesh=mesh,
        compiler_params=pltpu.CompilerParams(needs_layout_passes=False),
        scratch_types=[
            pltpu.VMEM((DNCH, DCH), jnp.int32),
            pltpu.VMEM((NP,), jnp.float32),
            pltpu.VMEM((RPT,), jnp.float32),
            pltpu.VMEM((NS, RPT), jnp.float32),
            pltpu.VMEM_SHARED((NS, NP), jnp.float32),
        ],
    )
    agg = pl.kernel(
        _agg_body,
        out_type=jax.ShapeDtypeStruct((NP, D), jnp.float32),
        mesh=mesh,
        compiler_params=pltpu.CompilerParams(needs_layout_passes=False),
        scratch_types=[
            pltpu.VMEM((NCHH, CH), jnp.int32),
            pltpu.VMEM((NCHH, CH), jnp.int32),
            pltpu.VMEM((CH, D), jnp.float32),
            pltpu.VMEM((CH, D), jnp.float32),
            pltpu.VMEM((CH, D), jnp.float32),
            pltpu.VMEM((CH, D), jnp.float32),
            pltpu.VMEM_SHARED((HALFP, D), jnp.float32),
            pltpu.SemaphoreType.DMA,
            pltpu.SemaphoreType.DMA,
            pltpu.SemaphoreType.DMA,
            pltpu.SemaphoreType.DMA,
        ],
    )
    return deg, agg


def kernel(x, edge_index, batch, W1, b1, W2, b2, Wc, bc):
    _deg_kernel, _agg_kernel = _sc_kernels()
    _prep_call, _mid_call, _fin_call = _tc_kernels()
    xp = jnp.pad(x, ((0, NP - N), (0, 0)))
    src2 = jnp.pad(edge_index[0], (0, EPAD - E)).reshape(NS, NCH, CH)
    dst2 = jnp.pad(edge_index[1], (0, EPAD - E),
                   constant_values=NP).reshape(NS, NCH, CH)
    dst3d = edge_index[1].reshape(NW, DNCH, DCH)
    batch_row = jnp.pad(batch, (0, NP - N), constant_values=G).reshape(1, NP)
    batch_row = batch_row.astype(jnp.int32)

    degp = _deg_kernel(dst3d)
    p0 = degp[0].reshape(NP, 1)
    p1 = degp[1].reshape(NP, 1)

    t1 = _prep_call(xp, W1, p0, p1)
    a1 = _agg_kernel(src2, dst2, t1)
    t2 = _mid_call(a1, t1, p0, p1, b1.reshape(1, D), W2)
    a2 = _agg_kernel(src2, dst2, t2)
    out = _fin_call(a2, t2, p0, p1, b2.reshape(1, D),
                    batch_row, Wc, bc.reshape(1, 2))
    return out


# speedup vs baseline: 1.0022x; 1.0022x over previous
import functools

import jax
import jax.numpy as jnp
from jax import lax
from jax.experimental import pallas as pl
from jax.experimental.pallas import tpu as pltpu
from jax.experimental.pallas import tpu_sc as plsc

N = 10000
E = 320000
D = 128
G = 16
NP = 10240

NC = 2
NS = 16
NW = NC * NS
L = 16

EPW = E // NW
DCH = 80
DNCH = EPW // DCH
RPT = NP // NS

CH = 80
NCH = 256
NCHH = NCH // 2
EPAD = NS * NCH * CH
HALF = NP // NC
TRASH = HALF
HALFP = HALF + CH
RPC = HALF // NS


def _deg_body(dst_hbm, out_hbm, didx, hist, outst, red, shared):
    c = lax.axis_index("c")
    s = lax.axis_index("s")
    w = s * NC + c
    pltpu.sync_copy(dst_hbm.at[w], didx)

    zero16 = jnp.zeros((L,), jnp.float32)
    ones16 = jnp.full((L,), 1.0, jnp.float32)

    def _zero(i, carry):
        hist[pl.ds(i * L, L)] = zero16
        return carry
    lax.fori_loop(0, NP // L, _zero, 0)

    def _count(r, carry):
        def _inner(cc, carry2):
            idx = didx[r, pl.ds(cc * L, L)]
            plsc.addupdate_scatter(hist, [idx], ones16)
            return carry2
        return lax.fori_loop(0, DCH // L, _inner, carry)
    lax.fori_loop(0, DNCH, _count, 0)

    pltpu.sync_copy(hist, shared.at[s])
    plsc.subcore_barrier()
    base = s * RPT
    pltpu.sync_copy(shared.at[:, pl.ds(base, RPT)], red)

    def _reduce(j, carry):
        acc = jnp.zeros((L,), jnp.float32)
        for r in range(NS):
            acc = acc + red[r, pl.ds(j * L, L)]
        outst[pl.ds(j * L, L)] = acc
        return carry
    lax.fori_loop(0, RPT // L, _reduce, 0)
    pltpu.sync_copy(outst, out_hbm.at[c, pl.ds(base, RPT)])


def _agg_body(src_hbm, dst_hbm, tab_hbm, out_hbm,
              sidx, didx, rows_a, rows_b, rows_c, rows_d, acc,
              sem_a, sem_b, sem_c, sem_d):
    c = lax.axis_index("c")
    s = lax.axis_index("s")
    base = s * RPC

    zero16 = jnp.zeros((L,), jnp.float32)

    def _zfill(i, carry):
        r = i // (D // L)
        col = (i % (D // L)) * L
        rows_a[r, pl.ds(col, L)] = zero16
        return carry
    lax.fori_loop(0, CH * (D // L), _zfill, 0)

    for i in range(RPC // CH):
        pltpu.sync_copy(rows_a, acc.at[pl.ds(base + i * CH, CH)])

    @pl.when(s == 0)
    def _():
        pltpu.sync_copy(rows_a, acc.at[pl.ds(HALF, CH)])

    plsc.subcore_barrier()

    lo = c * HALF

    for p in range(2):
        pltpu.sync_copy(src_hbm.at[s, pl.ds(p * NCHH, NCHH)], sidx)
        pltpu.sync_copy(dst_hbm.at[s, pl.ds(p * NCHH, NCHH)], didx)

        def _remap(r, carry):
            for cc in range(CH // L):
                g = didx[r, pl.ds(cc * L, L)] - lo
                ok = (g >= 0) & (g < HALF)
                didx[r, pl.ds(cc * L, L)] = jnp.where(ok, g, TRASH)
            return carry
        lax.fori_loop(0, NCHH, _remap, 0)

        pltpu.async_copy(tab_hbm.at[sidx.at[0]], rows_a, sem_a)
        pltpu.async_copy(tab_hbm.at[sidx.at[1]], rows_b, sem_b)
        pltpu.async_copy(tab_hbm.at[sidx.at[2]], rows_c, sem_c)

        def _quad(t, carry):
            j = 4 * t
            pltpu.make_async_copy(tab_hbm.at[sidx.at[j]], rows_a, sem_a).wait()
            pltpu.async_copy(tab_hbm.at[sidx.at[j + 3]], rows_d, sem_d)
            pltpu.sync_copy(rows_a, acc.at[didx.at[j]], add=True)

            pltpu.make_async_copy(tab_hbm.at[sidx.at[j + 1]], rows_b,
                                  sem_b).wait()
            @pl.when(j + 4 < NCHH)
            def _():
                pltpu.async_copy(tab_hbm.at[sidx.at[j + 4]], rows_a, sem_a)
            pltpu.sync_copy(rows_b, acc.at[didx.at[j + 1]], add=True)

            pltpu.make_async_copy(tab_hbm.at[sidx.at[j + 2]], rows_c,
                                  sem_c).wait()
            @pl.when(j + 5 < NCHH)
            def _():
                pltpu.async_copy(tab_hbm.at[sidx.at[j + 5]], rows_b, sem_b)
            pltpu.sync_copy(rows_c, acc.at[didx.at[j + 2]], add=True)

            pltpu.make_async_copy(tab_hbm.at[sidx.at[j + 3]], rows_d,
                                  sem_d).wait()
            @pl.when(j + 6 < NCHH)
            def _():
                pltpu.async_copy(tab_hbm.at[sidx.at[j + 6]], rows_c, sem_c)
            pltpu.sync_copy(rows_d, acc.at[didx.at[j + 3]], add=True)
            return carry
        lax.fori_loop(0, NCHH // 4, _quad, 0)

    plsc.subcore_barrier()
    pltpu.sync_copy(acc.at[pl.ds(base, RPC)],
                    out_hbm.at[pl.ds(c * HALF + base, RPC)])


BLK = 2048
NBLK = NP // BLK


def _prep_body(x_ref, w1_ref, p0_ref, p1_ref, o_ref):
    dinv = lax.rsqrt(1.0 + p0_ref[...] + p1_ref[...])
    h = jnp.dot(x_ref[...], w1_ref[...], preferred_element_type=jnp.float32)
    o_ref[...] = h * dinv


def _mid_body(a_ref, t1_ref, p0_ref, p1_ref, b1_ref, w2_ref, o_ref):
    dinv = lax.rsqrt(1.0 + p0_ref[...] + p1_ref[...])
    z = dinv * (a_ref[...] + t1_ref[...]) + b1_ref[...]
    z = jnp.maximum(z, 0.0)
    o_ref[...] = jnp.dot(z, w2_ref[...], preferred_element_type=jnp.float32) * dinv


def _fin_body(a_ref, t2_ref, p0_ref, p1_ref, b2_ref, br_ref,
              wc_ref, bc_ref, o_ref, sums, cnt):
    i = pl.program_id(0)

    @pl.when(i == 0)
    def _():
        sums[...] = jnp.zeros_like(sums)
        cnt[...] = jnp.zeros_like(cnt)

    dinv = lax.rsqrt(1.0 + p0_ref[...] + p1_ref[...])
    z = dinv * (a_ref[...] + t2_ref[...]) + b2_ref[...]
    ids = lax.broadcasted_iota(jnp.int32, (G, BLK), 0)
    oh = (ids == br_ref[...]).astype(jnp.float32)
    sums[...] += jnp.dot(oh, z, preferred_element_type=jnp.float32)
    cnt[...] += jnp.sum(oh, axis=1, keepdims=True)

    @pl.when(i == pl.num_programs(0) - 1)
    def _():
        pooled = sums[...] / jnp.maximum(cnt[...], 1.0)
        o_ref[...] = (jnp.dot(pooled, wc_ref[...],
                              preferred_element_type=jnp.float32) + bc_ref[...])


def _row_spec(width):
    return pl.BlockSpec((BLK, width), lambda i: (i, 0))


def _const_spec(shape):
    return pl.BlockSpec(shape, lambda i: tuple(0 for _ in shape))


@functools.lru_cache(maxsize=2)
def _tc_kernels(interpret: bool = False):
    prep = pl.pallas_call(
        _prep_body,
        grid=(NBLK,),
        in_specs=[_row_spec(D), _const_spec((D, D)), _row_spec(1),
                  _row_spec(1)],
        out_specs=_row_spec(D),
        out_shape=jax.ShapeDtypeStruct((NP, D), jnp.float32),
        interpret=interpret,
    )
    mid = pl.pallas_call(
        _mid_body,
        grid=(NBLK,),
        in_specs=[_row_spec(D), _row_spec(D), _row_spec(1), _row_spec(1),
                  _const_spec((1, D)), _const_spec((D, D))],
        out_specs=_row_spec(D),
        out_shape=jax.ShapeDtypeStruct((NP, D), jnp.float32),
        interpret=interpret,
    )
    fin = pl.pallas_call(
        _fin_body,
        grid=(NBLK,),
        in_specs=[_row_spec(D), _row_spec(D), _row_spec(1), _row_spec(1),
                  _const_spec((1, D)),
                  pl.BlockSpec((1, BLK), lambda i: (0, i)),
                  _const_spec((D, 2)), _const_spec((1, 2))],
        out_specs=_const_spec((G, 2)),
        out_shape=jax.ShapeDtypeStruct((G, 2), jnp.float32),
        scratch_shapes=[pltpu.VMEM((G, D), jnp.float32),
                        pltpu.VMEM((G, 1), jnp.float32)],
        interpret=interpret,
    )
    return prep, mid, fin


@functools.lru_cache(maxsize=1)
def _sc_kernels():
    mesh = plsc.VectorSubcoreMesh(core_axis_name="c", subcore_axis_name="s",
                                  num_cores=NC, num_subcores=NS)
    deg = pl.kernel(
        _deg_body,
        out_type=jax.ShapeDtypeStruct((NC, NP), jnp.float32),
        mesh=mesh,
        compiler_params=pltpu.CompilerParams(needs_layout_passes=False),
        scratch_types=[
            pltpu.VMEM((DNCH, DCH), jnp.int32),
            pltpu.VMEM((NP,), jnp.float32),
            pltpu.VMEM((RPT,), jnp.float32),
            pltpu.VMEM((NS, RPT), jnp.float32),
            pltpu.VMEM_SHARED((NS, NP), jnp.float32),
        ],
    )
    agg = pl.kernel(
        _agg_body,
        out_type=jax.ShapeDtypeStruct((NP, D), jnp.float32),
        mesh=mesh,
        compiler_params=pltpu.CompilerParams(needs_layout_passes=False),
        scratch_types=[
            pltpu.VMEM((NCHH, CH), jnp.int32),
            pltpu.VMEM((NCHH, CH), jnp.int32),
            pltpu.VMEM((CH, D), jnp.float32),
            pltpu.VMEM((CH, D), jnp.float32),
            pltpu.VMEM((CH, D), jnp.float32),
            pltpu.VMEM((CH, D), jnp.float32),
            pltpu.VMEM_SHARED((HALFP, D), jnp.float32),
            pltpu.SemaphoreType.DMA,
            pltpu.SemaphoreType.DMA,
            pltpu.SemaphoreType.DMA,
            pltpu.SemaphoreType.DMA,
        ],
    )
    return deg, agg


def kernel(x, edge_index, batch, W1, b1, W2, b2, Wc, bc):
    _deg_kernel, _agg_kernel = _sc_kernels()
    _prep_call, _mid_call, _fin_call = _tc_kernels()
    xp = jnp.pad(x, ((0, NP - N), (0, 0)))
    src2 = jnp.pad(edge_index[0], (0, EPAD - E)).reshape(NS, NCH, CH)
    dst2 = jnp.pad(edge_index[1], (0, EPAD - E),
                   constant_values=NP).reshape(NS, NCH, CH)
    dst3d = edge_index[1].reshape(NW, DNCH, DCH)
    batch_row = jnp.pad(batch, (0, NP - N), constant_values=G).reshape(1, NP)
    batch_row = batch_row.astype(jnp.int32)

    degp = _deg_kernel(dst3d)
    p0 = degp[0].reshape(NP, 1)
    p1 = degp[1].reshape(NP, 1)

    t1 = _prep_call(xp, W1, p0, p1)
    a1 = _agg_kernel(src2, dst2, t1)
    t2 = _mid_call(a1, t1, p0, p1, b1.reshape(1, D), W2)
    a2 = _agg_kernel(src2, dst2, t2)
    out = _fin_call(a2, t2, p0, p1, b2.reshape(1, D),
                    batch_row, Wc, bc.reshape(1, 2))
    return out


# speedup vs baseline: 2.4665x; 2.4610x over previous
import functools

import jax
import jax.numpy as jnp
from jax import lax
from jax.experimental import pallas as pl
from jax.experimental.pallas import tpu as pltpu
from jax.experimental.pallas import tpu_sc as plsc

N = 10000
E = 320000
D = 128
G = 16
NP = 10240

NC = 2
NS = 16
NW = NC * NS
L = 16

EPW = E // NW
DCH = 80
DNCH = EPW // DCH
RPT = NP // NS

EPS = E // NS
CH = 80
NCH = EPS // CH
HALF = NP // NC
TRASH = HALF
HALFP = HALF + CH
RPC = HALF // NS


def _deg_body(dst_hbm, out_hbm, didx, hist, outst, red, shared):
    c = lax.axis_index("c")
    s = lax.axis_index("s")
    w = s * NC + c
    pltpu.sync_copy(dst_hbm.at[w], didx)

    zero16 = jnp.zeros((L,), jnp.float32)
    ones16 = jnp.full((L,), 1.0, jnp.float32)

    def _zero(i, carry):
        hist[pl.ds(i * L, L)] = zero16
        return carry
    lax.fori_loop(0, NP // L, _zero, 0)

    def _count(r, carry):
        def _inner(cc, carry2):
            idx = didx[r, pl.ds(cc * L, L)]
            plsc.addupdate_scatter(hist, [idx], ones16)
            return carry2
        return lax.fori_loop(0, DCH // L, _inner, carry)
    lax.fori_loop(0, DNCH, _count, 0)

    pltpu.sync_copy(hist, shared.at[s])
    plsc.subcore_barrier()
    base = s * RPT
    pltpu.sync_copy(shared.at[:, pl.ds(base, RPT)], red)

    def _reduce(j, carry):
        acc = jnp.zeros((L,), jnp.float32)
        for r in range(NS):
            acc = acc + red[r, pl.ds(j * L, L)]
        outst[pl.ds(j * L, L)] = acc
        return carry
    lax.fori_loop(0, RPT // L, _reduce, 0)
    pltpu.sync_copy(outst, out_hbm.at[c, pl.ds(base, RPT)])


def _agg_body(src_hbm, dst_hbm, tab_hbm, out_hbm,
              sidx, didx, rows_a, rows_b, acc, sem_a, sem_b):
    c = lax.axis_index("c")
    s = lax.axis_index("s")
    base = s * RPC

    zero16 = jnp.zeros((L,), jnp.float32)

    def _zfill(i, carry):
        r = i // (D // L)
        col = (i % (D // L)) * L
        rows_a[r, pl.ds(col, L)] = zero16
        return carry
    lax.fori_loop(0, CH * (D // L), _zfill, 0)

    for i in range(RPC // CH):
        pltpu.sync_copy(rows_a, acc.at[pl.ds(base + i * CH, CH)])

    @pl.when(s == 0)
    def _():
        pltpu.sync_copy(rows_a, acc.at[pl.ds(HALF, CH)])

    pltpu.sync_copy(src_hbm.at[s], sidx)
    pltpu.sync_copy(dst_hbm.at[s], didx)

    lo = c * HALF

    def _remap(r, carry):
        for cc in range(CH // L):
            g = didx[r, pl.ds(cc * L, L)] - lo
            ok = (g >= 0) & (g < HALF)
            didx[r, pl.ds(cc * L, L)] = jnp.where(ok, g, TRASH)
        return carry
    lax.fori_loop(0, NCH, _remap, 0)

    plsc.subcore_barrier()

    pltpu.async_copy(tab_hbm.at[sidx.at[0]], rows_a, sem_a)

    def _pair(t, carry):
        j = 2 * t
        pltpu.make_async_copy(tab_hbm.at[sidx.at[j]], rows_a, sem_a).wait()
        pltpu.async_copy(tab_hbm.at[sidx.at[j + 1]], rows_b, sem_b)
        pltpu.sync_copy(rows_a, acc.at[didx.at[j]], add=True)
        pltpu.make_async_copy(tab_hbm.at[sidx.at[j + 1]], rows_b, sem_b).wait()
        @pl.when(t + 1 < NCH // 2)
        def _():
            pltpu.async_copy(tab_hbm.at[sidx.at[j + 2]], rows_a, sem_a)
        pltpu.sync_copy(rows_b, acc.at[didx.at[j + 1]], add=True)
        return carry
    lax.fori_loop(0, NCH // 2, _pair, 0)

    plsc.subcore_barrier()
    pltpu.sync_copy(acc.at[pl.ds(base, RPC)],
                    out_hbm.at[pl.ds(c * HALF + base, RPC)])


BLK = 2048
NBLK = NP // BLK


def _prep_body(x_ref, w1_ref, p0_ref, p1_ref, o_ref):
    dinv = lax.rsqrt(1.0 + p0_ref[...] + p1_ref[...])
    h = jnp.dot(x_ref[...], w1_ref[...], preferred_element_type=jnp.float32)
    o_ref[...] = h * dinv


def _mid_body(a_ref, t1_ref, p0_ref, p1_ref, b1_ref, w2_ref, o_ref):
    dinv = lax.rsqrt(1.0 + p0_ref[...] + p1_ref[...])
    z = dinv * (a_ref[...] + t1_ref[...]) + b1_ref[...]
    z = jnp.maximum(z, 0.0)
    o_ref[...] = jnp.dot(z, w2_ref[...], preferred_element_type=jnp.float32) * dinv


def _fin_body(a_ref, t2_ref, p0_ref, p1_ref, b2_ref, br_ref,
              wc_ref, bc_ref, o_ref, sums, cnt):
    i = pl.program_id(0)

    @pl.when(i == 0)
    def _():
        sums[...] = jnp.zeros_like(sums)
        cnt[...] = jnp.zeros_like(cnt)

    dinv = lax.rsqrt(1.0 + p0_ref[...] + p1_ref[...])
    z = dinv * (a_ref[...] + t2_ref[...]) + b2_ref[...]
    ids = lax.broadcasted_iota(jnp.int32, (G, BLK), 0)
    oh = (ids == br_ref[...]).astype(jnp.float32)
    sums[...] += jnp.dot(oh, z, preferred_element_type=jnp.float32)
    cnt[...] += jnp.sum(oh, axis=1, keepdims=True)

    @pl.when(i == pl.num_programs(0) - 1)
    def _():
        pooled = sums[...] / jnp.maximum(cnt[...], 1.0)
        o_ref[...] = (jnp.dot(pooled, wc_ref[...],
                              preferred_element_type=jnp.float32) + bc_ref[...])


def _row_spec(width):
    return pl.BlockSpec((BLK, width), lambda i: (i, 0))


def _const_spec(shape):
    return pl.BlockSpec(shape, lambda i: tuple(0 for _ in shape))


@functools.lru_cache(maxsize=2)
def _tc_kernels(interpret: bool = False):
    prep = pl.pallas_call(
        _prep_body,
        grid=(NBLK,),
        in_specs=[_row_spec(D), _const_spec((D, D)), _row_spec(1),
                  _row_spec(1)],
        out_specs=_row_spec(D),
        out_shape=jax.ShapeDtypeStruct((NP, D), jnp.float32),
        interpret=interpret,
    )
    mid = pl.pallas_call(
        _mid_body,
        grid=(NBLK,),
        in_specs=[_row_spec(D), _row_spec(D), _row_spec(1), _row_spec(1),
                  _const_spec((1, D)), _const_spec((D, D))],
        out_specs=_row_spec(D),
        out_shape=jax.ShapeDtypeStruct((NP, D), jnp.float32),
        interpret=interpret,
    )
    fin = pl.pallas_call(
        _fin_body,
        grid=(NBLK,),
        in_specs=[_row_spec(D), _row_spec(D), _row_spec(1), _row_spec(1),
                  _const_spec((1, D)),
                  pl.BlockSpec((1, BLK), lambda i: (0, i)),
                  _const_spec((D, 2)), _const_spec((1, 2))],
        out_specs=_const_spec((G, 2)),
        out_shape=jax.ShapeDtypeStruct((G, 2), jnp.float32),
        scratch_shapes=[pltpu.VMEM((G, D), jnp.float32),
                        pltpu.VMEM((G, 1), jnp.float32)],
        interpret=interpret,
    )
    return prep, mid, fin


@functools.lru_cache(maxsize=1)
def _sc_kernels():
    mesh = plsc.VectorSubcoreMesh(core_axis_name="c", subcore_axis_name="s",
                                  num_cores=NC, num_subcores=NS)
    deg = pl.kernel(
        _deg_body,
        out_type=jax.ShapeDtypeStruct((NC, NP), jnp.float32),
        mesh=mesh,
        compiler_params=pltpu.CompilerParams(needs_layout_passes=False),
        scratch_types=[
            pltpu.VMEM((DNCH, DCH), jnp.int32),
            pltpu.VMEM((NP,), jnp.float32),
            pltpu.VMEM((RPT,), jnp.float32),
            pltpu.VMEM((NS, RPT), jnp.float32),
            pltpu.VMEM_SHARED((NS, NP), jnp.float32),
        ],
    )
    agg = pl.kernel(
        _agg_body,
        out_type=jax.ShapeDtypeStruct((NP, D), jnp.float32),
        mesh=mesh,
        compiler_params=pltpu.CompilerParams(needs_layout_passes=False),
        scratch_types=[
            pltpu.VMEM((NCH, CH), jnp.int32),
            pltpu.VMEM((NCH, CH), jnp.int32),
            pltpu.VMEM((CH, D), jnp.float32),
            pltpu.VMEM((CH, D), jnp.float32),
            pltpu.VMEM_SHARED((HALFP, D), jnp.float32),
            pltpu.SemaphoreType.DMA,
            pltpu.SemaphoreType.DMA,
        ],
    )
    return deg, agg


def kernel(x, edge_index, batch, W1, b1, W2, b2, Wc, bc):
    _deg_kernel, _agg_kernel = _sc_kernels()
    _prep_call, _mid_call, _fin_call = _tc_kernels()
    xp = jnp.pad(x, ((0, NP - N), (0, 0)))
    src2 = edge_index[0].reshape(NS, NCH, CH)
    dst2 = edge_index[1].reshape(NS, NCH, CH)
    dst3d = edge_index[1].reshape(NW, DNCH, DCH)
    batch_row = jnp.pad(batch, (0, NP - N), constant_values=G).reshape(1, NP)
    batch_row = batch_row.astype(jnp.int32)

    degp = _deg_kernel(dst3d)
    p0 = degp[0].reshape(NP, 1)
    p1 = degp[1].reshape(NP, 1)

    t1 = _prep_call(xp, W1, p0, p1)
    a1 = _agg_kernel(src2, dst2, t1)
    t2 = _mid_call(a1, t1, p0, p1, b1.reshape(1, D), W2)
    a2 = _agg_kernel(src2, dst2, t2)
    out = _fin_call(a2, t2, p0, p1, b2.reshape(1, D),
                    batch_row, Wc, bc.reshape(1, 2))
    return out
